# baseline (speedup 1.0000x reference)
import jax
import jax.numpy as jnp
from jax.experimental import pallas as pl
from jax.experimental.pallas import tpu as pltpu

_TRANSFORMS = [jnp.tanh, jax.nn.relu, jax.nn.gelu, jax.nn.sigmoid]

_I0, _I1 = 1, 2

_BLOCK_ROWS = 8


def _body(x_ref, o_ref):
    o_ref[...] = _TRANSFORMS[_I1](_TRANSFORMS[_I0](x_ref[...]))


def kernel(x):
    rows, cols = x.shape
    return pl.pallas_call(
        _body,
        grid=(rows // _BLOCK_ROWS,),
        in_specs=[pl.BlockSpec((_BLOCK_ROWS, cols), lambda g: (g, 0))],
        out_specs=pl.BlockSpec((_BLOCK_ROWS, cols), lambda g: (g, 0)),
        out_shape=jax.ShapeDtypeStruct(x.shape, x.dtype),
        compiler_params=pltpu.CompilerParams(
            dimension_semantics=("parallel",)
        ),
    )(x)
